# baseline (speedup 1.0000x reference)
import jax
import jax.numpy as jnp
from jax.experimental import pallas as pl
from jax.experimental.pallas import tpu as pltpu

B = 16384
D = 32
G = 8
NSTEP = B // G


def _dot_kernel(users_ref, items_ref, *refs):
    o_ref = refs[-1]
    u_blocks = refs[:G]
    v_blocks = refs[G:2 * G]
    g = pl.program_id(0)
    sub = jax.lax.broadcasted_iota(jnp.int32, (G, 1), 0)
    urows = []
    vrows = []
    for j in range(G):
        ru = users_ref[g * G + j] % G
        rv = items_ref[g * G + j] % G
        umask = (sub == ru).astype(jnp.float32)
        vmask = (sub == rv).astype(jnp.float32)
        urows.append(jnp.sum(u_blocks[j][...] * umask, axis=0, keepdims=True))
        vrows.append(jnp.sum(v_blocks[j][...] * vmask, axis=0, keepdims=True))
    u = jnp.concatenate(urows, axis=0)
    v = jnp.concatenate(vrows, axis=0)
    dots = jnp.sum(u * v, axis=1).reshape(1, G)
    o_ref[pl.ds(g, 1), :] = dots


def kernel(data, user_factors, item_factors):
    users = data[:, 0].astype(jnp.int32)
    items = data[:, 1].astype(jnp.int32)

    def mk_spec(idx_pos, j):
        if idx_pos == 0:
            return pl.BlockSpec(
                (G, D), lambda g, users, items, j=j: (users[g * G + j] // G, 0))
        return pl.BlockSpec(
            (G, D), lambda g, users, items, j=j: (items[g * G + j] // G, 0))

    in_specs = ([mk_spec(0, j) for j in range(G)]
                + [mk_spec(1, j) for j in range(G)])
    out = pl.pallas_call(
        _dot_kernel,
        grid_spec=pltpu.PrefetchScalarGridSpec(
            num_scalar_prefetch=2,
            grid=(NSTEP,),
            in_specs=in_specs,
            out_specs=pl.BlockSpec((NSTEP, G), lambda g, users, items: (0, 0)),
        ),
        out_shape=jax.ShapeDtypeStruct((NSTEP, G), jnp.float32),
    )(users, items, *([user_factors] * G), *([item_factors] * G))
    return out.reshape(B)
